# baseline (speedup 1.0000x reference)
import functools

import jax
import jax.numpy as jnp
from jax import lax
from jax.experimental import pallas as pl
from jax.experimental.pallas import tpu as pltpu
from jax.experimental.pallas import tpu_sc as plsc

_REP = 128
_LANES = 16


def kernel(mask, embed_weight):
    B, L = mask.shape
    D = embed_weight.shape[-1]
    info = plsc.get_sparse_core_info()
    n_cores, n_sub = info.num_cores, info.num_subcores
    n_workers = n_cores * n_sub
    rows_per_w = L // n_workers
    n_chunks = B // _REP

    mesh = plsc.VectorSubcoreMesh(core_axis_name="c", subcore_axis_name="s")

    @functools.partial(
        pl.kernel,
        out_type=jax.ShapeDtypeStruct((L, B, D), embed_weight.dtype),
        mesh=mesh,
        scratch_types=[
            pltpu.VMEM((1, D), embed_weight.dtype),
            pltpu.VMEM((_REP, D), embed_weight.dtype),
            pltpu.SemaphoreType.DMA,
        ],
    )
    def sc_embed(table_hbm, out_hbm, row_v, rep_v, sem):
        wid = lax.axis_index("s") * n_cores + lax.axis_index("c")

        def do_row(j, carry):
            l = wid * rows_per_w + j
            pltpu.sync_copy(table_hbm.at[pl.ds(l, 1)], row_v)

            def fill(i, c2):
                for v in range(D // _LANES):
                    rep_v[i, pl.ds(v * _LANES, _LANES)] = row_v[
                        0, pl.ds(v * _LANES, _LANES)
                    ]
                return c2

            lax.fori_loop(0, _REP, fill, 0)

            def fire(i, c2):
                pltpu.make_async_copy(
                    rep_v, out_hbm.at[l, pl.ds(i * _REP, _REP)], sem
                ).start()
                return c2

            lax.fori_loop(0, n_chunks, fire, 0)

            def drain(i, c2):
                pltpu.make_async_copy(
                    rep_v, out_hbm.at[l, pl.ds(i * _REP, _REP)], sem
                ).wait()
                return c2

            lax.fori_loop(0, n_chunks, drain, 0)
            return carry

        lax.fori_loop(0, rows_per_w, do_row, 0)

    return sc_embed(embed_weight)
